# baseline (speedup 1.0000x reference)
import functools
import math

import jax
import jax.numpy as jnp
from jax.experimental import pallas as pl
from jax.experimental.pallas import tpu as pltpu

_F = 26
_C = 1000
_D = 16
_K = _F * _C

_NCALLS = 8


def _mm_body(z_ref, w_ref, o_ref):
    o_ref[:] = jnp.dot(
        z_ref[:].astype(jnp.bfloat16),
        w_ref[:],
        preferred_element_type=jnp.float32,
    )


@functools.partial(jax.jit, static_argnums=(2,))
def _chunk_call(z_chunk, w, bm):
    return pl.pallas_call(
        _mm_body,
        in_specs=[
            pl.BlockSpec(memory_space=pltpu.MemorySpace.VMEM),
            pl.BlockSpec(memory_space=pltpu.MemorySpace.VMEM),
        ],
        out_specs=pl.BlockSpec(memory_space=pltpu.MemorySpace.VMEM),
        out_shape=jax.ShapeDtypeStruct((bm, _D), jnp.float32),
    )(z_chunk, w)


def kernel(z, codebook):
    batch_shape = z.shape[:-1]
    m = math.prod(batch_shape)
    z2 = z.reshape(m, _K)
    w = codebook.reshape(_K, _D).astype(jnp.bfloat16)

    bm = m // _NCALLS
    outs = [
        _chunk_call(jax.lax.slice(z2, (c * bm, 0), ((c + 1) * bm, _K)), w, bm)
        for c in range(_NCALLS)
    ]
    out = jnp.concatenate(outs, axis=0)
    return out.reshape(*batch_shape, _D)


# speedup vs baseline: 1.7950x; 1.7950x over previous
import math

import jax
import jax.numpy as jnp
from jax.experimental import pallas as pl
from jax.experimental.pallas import tpu as pltpu

_F = 26
_C = 1000
_D = 16
_K = _F * _C

_KALN = 25984
_CHUNK = 896
_NCH = _KALN // _CHUNK
_NBUF = 4


def _mm_body(z_hbm, w_ref, tail_ref, wtail_ref, o_ref, buf, sems):
    i = pl.program_id(0)

    def copy(c, slot):
        return pltpu.make_async_copy(
            z_hbm.at[:, pl.ds(c * _CHUNK, _CHUNK)],
            buf.at[slot],
            sems.at[slot],
        )

    @pl.when(i == 0)
    def _warmup():
        for s in range(_NBUF - 1):
            copy(s, s).start(priority=s % 2)

    nxt = i + _NBUF - 1

    @pl.when(jnp.logical_and(nxt < _NCH, jax.lax.rem(nxt, 2) == 0))
    def _prefetch_t0():
        copy(nxt, jax.lax.rem(nxt, _NBUF)).start(priority=0)

    @pl.when(jnp.logical_and(nxt < _NCH, jax.lax.rem(nxt, 2) == 1))
    def _prefetch_t1():
        copy(nxt, jax.lax.rem(nxt, _NBUF)).start(priority=1)

    slot = jax.lax.rem(i, _NBUF)
    copy(i, slot).wait()

    part = jnp.dot(
        buf[slot],
        w_ref[pl.ds(i * _CHUNK, _CHUNK), :],
        preferred_element_type=jnp.float32,
    )

    @pl.when(i == 0)
    def _init():
        o_ref[:] = part + jnp.dot(
            tail_ref[:], wtail_ref[:], preferred_element_type=jnp.float32
        )

    @pl.when(i > 0)
    def _acc():
        o_ref[:] += part


def kernel(z, codebook):
    batch_shape = z.shape[:-1]
    m = math.prod(batch_shape)
    z2 = z.reshape(m, _K)
    w = codebook.reshape(_K, _D)
    z_tail = z2[:, _KALN:]
    w_tail = w[_KALN:, :]

    out = pl.pallas_call(
        _mm_body,
        grid=(_NCH,),
        in_specs=[
            pl.BlockSpec(memory_space=pltpu.MemorySpace.HBM),
            pl.BlockSpec((_K, _D), lambda i: (0, 0)),
            pl.BlockSpec((m, _K - _KALN), lambda i: (0, 0)),
            pl.BlockSpec((_K - _KALN, _D), lambda i: (0, 0)),
        ],
        out_specs=pl.BlockSpec((m, _D), lambda i: (0, 0)),
        out_shape=jax.ShapeDtypeStruct((m, _D), jnp.float32),
        scratch_shapes=[
            pltpu.VMEM((_NBUF, m, _CHUNK), jnp.float32),
            pltpu.SemaphoreType.DMA((_NBUF,)),
        ],
    )(z2, w, z_tail, w_tail)
    return out.reshape(*batch_shape, _D)
